# baseline (device time: 452928 ns/iter reference)
import jax
import jax.numpy as jnp
from jax import lax
from jax.experimental import pallas as pl
from jax.experimental.pallas import tpu as pltpu

M_GLOBAL = 8192
M_HALF = 4096
D = 4096
CH = 256
K = M_HALF // CH


def kernel(partial, gamma):
    partial2 = partial.reshape(M_GLOBAL, D)
    gamma2 = gamma.reshape(1, D)

    def body(partial_ref, gamma_ref, out_ref, recv_ref,
             in_vmem, send_vmem, b_vmem, o_vmem,
             sem_in, sem_a, sem_b, sem_o, send_sems, recv_sems):
        a_vmem = in_vmem
        x = lax.axis_index("x")
        y = lax.axis_index("y")
        z = lax.axis_index("z")
        peer = (x, y, 1 - z)

        barrier_sem = pltpu.get_barrier_semaphore()
        pl.semaphore_signal(
            barrier_sem, inc=1,
            device_id=peer, device_id_type=pl.DeviceIdType.MESH,
        )
        pl.semaphore_wait(barrier_sem, 1)

        other_off = (1 - z) * M_HALF
        mine_off = z * M_HALF

        cp_in = [None, None]
        cp_in[0] = pltpu.make_async_copy(
            partial_ref.at[pl.ds(other_off, CH), :],
            in_vmem.at[0], sem_in.at[0])
        cp_in[0].start()
        rdmas = []
        for c in range(K):
            si = c % 2
            if c + 1 < K:
                cp_in[1 - si] = pltpu.make_async_copy(
                    partial_ref.at[pl.ds(other_off + (c + 1) * CH, CH), :],
                    in_vmem.at[1 - si], sem_in.at[1 - si])
                cp_in[1 - si].start()
            cp_in[si].wait()
            if c >= 2:
                rdmas[c - 2].wait_send()
            send_vmem[si] = in_vmem[si].astype(jnp.bfloat16)
            rdma = pltpu.make_async_remote_copy(
                src_ref=send_vmem.at[si],
                dst_ref=recv_ref.at[pl.ds(c * CH, CH), :],
                send_sem=send_sems.at[c],
                recv_sem=recv_sems.at[c],
                device_id=peer,
                device_id_type=pl.DeviceIdType.MESH,
            )
            rdma.start()
            rdmas.append(rdma)

        cp_a = [None, None]
        cp_a[0] = pltpu.make_async_copy(
            partial_ref.at[pl.ds(mine_off, CH), :], a_vmem.at[0], sem_a.at[0])
        cp_a[0].start()
        cp_o = [None, None]
        for c in range(K):
            s = c % 2
            if c + 1 < K:
                cp_a[1 - s] = pltpu.make_async_copy(
                    partial_ref.at[pl.ds(mine_off + (c + 1) * CH, CH), :],
                    a_vmem.at[1 - s], sem_a.at[1 - s])
                cp_a[1 - s].start()
            rdmas[c].wait_recv()
            cp_b = pltpu.make_async_copy(
                recv_ref.at[pl.ds(c * CH, CH), :], b_vmem.at[s], sem_b.at[s])
            cp_b.start()
            cp_b.wait()
            cp_a[s].wait()
            if cp_o[s] is not None:
                cp_o[s].wait()
            ysum = a_vmem[s] + b_vmem[s].astype(jnp.float32)
            ms = jnp.mean(ysum * ysum, axis=1, keepdims=True)
            o_vmem[s] = ysum * lax.rsqrt(ms + 1e-6) * gamma_ref[...]
            cp_o[s] = pltpu.make_async_copy(
                o_vmem.at[s], out_ref.at[pl.ds(c * CH, CH), :], sem_o.at[s])
            cp_o[s].start()

        for s in range(2):
            if cp_o[s] is not None:
                cp_o[s].wait()
        for c in range(K - 2, K):
            rdmas[c].wait_send()

    out, _recv = pl.pallas_call(
        body,
        out_shape=[
            jax.ShapeDtypeStruct((M_HALF, D), jnp.float32),
            jax.ShapeDtypeStruct((M_HALF, D), jnp.bfloat16),
        ],
        in_specs=[
            pl.BlockSpec(memory_space=pl.ANY),
            pl.BlockSpec(memory_space=pltpu.VMEM),
        ],
        out_specs=[
            pl.BlockSpec(memory_space=pl.ANY),
            pl.BlockSpec(memory_space=pl.ANY),
        ],
        scratch_shapes=[
            pltpu.VMEM((2, CH, D), jnp.float32),
            pltpu.VMEM((2, CH, D), jnp.bfloat16),
            pltpu.VMEM((2, CH, D), jnp.bfloat16),
            pltpu.VMEM((2, CH, D), jnp.float32),
            pltpu.SemaphoreType.DMA((2,)),
            pltpu.SemaphoreType.DMA((2,)),
            pltpu.SemaphoreType.DMA((2,)),
            pltpu.SemaphoreType.DMA((2,)),
            pltpu.SemaphoreType.DMA((K,)),
            pltpu.SemaphoreType.DMA((K,)),
        ],
        compiler_params=pltpu.CompilerParams(
            collective_id=0, vmem_limit_bytes=100 * 1024 * 1024
        ),
    )(partial2, gamma2)
    return out


# device time: 294156 ns/iter; 1.5398x vs baseline; 1.5398x over previous
import jax
import jax.numpy as jnp
from jax import lax
from jax.experimental import pallas as pl
from jax.experimental.pallas import tpu as pltpu

M_GLOBAL = 8192
M_HALF = 4096
D = 4096
CH = 512
K = M_HALF // CH


def kernel(partial, gamma):
    partial2 = partial.reshape(M_GLOBAL, D)
    gamma2 = gamma.reshape(1, D)

    def body(partial_ref, gamma_ref, out_ref, recvq_ref, recvs_ref,
             in_vmem, sendq_vmem, sends_vmem, bq_vmem, bs_vmem, o_vmem,
             sem_in, sem_a, sem_b, sem_bs, sem_o,
             sendq_sems, recvq_sems, sends_sems, recvs_sems):
        a_vmem = in_vmem
        x = lax.axis_index("x")
        y = lax.axis_index("y")
        z = lax.axis_index("z")
        peer = (x, y, 1 - z)

        barrier_sem = pltpu.get_barrier_semaphore()
        pl.semaphore_signal(
            barrier_sem, inc=1,
            device_id=peer, device_id_type=pl.DeviceIdType.MESH,
        )
        pl.semaphore_wait(barrier_sem, 1)

        other_off = (1 - z) * M_HALF
        mine_off = z * M_HALF

        cp_in = [None, None]
        cp_in[0] = pltpu.make_async_copy(
            partial_ref.at[pl.ds(other_off, CH), :],
            in_vmem.at[0], sem_in.at[0])
        cp_in[0].start()
        rdmas_q = []
        rdmas_s = []
        for c in range(K):
            si = c % 2
            if c + 1 < K:
                cp_in[1 - si] = pltpu.make_async_copy(
                    partial_ref.at[pl.ds(other_off + (c + 1) * CH, CH), :],
                    in_vmem.at[1 - si], sem_in.at[1 - si])
                cp_in[1 - si].start()
            cp_in[si].wait()
            if c >= 2:
                rdmas_q[c - 2].wait_send()
                rdmas_s[c - 2].wait_send()
            xf = in_vmem[si]
            inv = 127.0 / jnp.maximum(
                jnp.max(jnp.abs(xf), axis=1, keepdims=True), 1e-20)
            sendq_vmem[si] = jnp.round(xf * inv).astype(jnp.int8)
            sends_vmem[si] = 1.0 / inv
            rq = pltpu.make_async_remote_copy(
                src_ref=sendq_vmem.at[si],
                dst_ref=recvq_ref.at[pl.ds(c * CH, CH), :],
                send_sem=sendq_sems.at[c],
                recv_sem=recvq_sems.at[c],
                device_id=peer,
                device_id_type=pl.DeviceIdType.MESH,
            )
            rq.start()
            rdmas_q.append(rq)
            rs = pltpu.make_async_remote_copy(
                src_ref=sends_vmem.at[si],
                dst_ref=recvs_ref.at[pl.ds(c * CH, CH), :],
                send_sem=sends_sems.at[c],
                recv_sem=recvs_sems.at[c],
                device_id=peer,
                device_id_type=pl.DeviceIdType.MESH,
            )
            rs.start()
            rdmas_s.append(rs)

        cp_a = [None, None]
        cp_a[0] = pltpu.make_async_copy(
            partial_ref.at[pl.ds(mine_off, CH), :], a_vmem.at[0], sem_a.at[0])
        cp_a[0].start()
        cp_o = [None, None]
        for c in range(K):
            s = c % 2
            if c + 1 < K:
                cp_a[1 - s] = pltpu.make_async_copy(
                    partial_ref.at[pl.ds(mine_off + (c + 1) * CH, CH), :],
                    a_vmem.at[1 - s], sem_a.at[1 - s])
                cp_a[1 - s].start()
            rdmas_q[c].wait_recv()
            rdmas_s[c].wait_recv()
            cp_b = pltpu.make_async_copy(
                recvq_ref.at[pl.ds(c * CH, CH), :], bq_vmem.at[s], sem_b.at[s])
            cp_bs = pltpu.make_async_copy(
                recvs_ref.at[pl.ds(c * CH, CH), :], bs_vmem.at[s],
                sem_bs.at[s])
            cp_b.start()
            cp_bs.start()
            cp_b.wait()
            cp_bs.wait()
            cp_a[s].wait()
            if cp_o[s] is not None:
                cp_o[s].wait()
            b = bq_vmem[s].astype(jnp.float32) * bs_vmem[s]
            ysum = a_vmem[s] + b
            ms = jnp.mean(ysum * ysum, axis=1, keepdims=True)
            o_vmem[s] = ysum * lax.rsqrt(ms + 1e-6) * gamma_ref[...]
            cp_o[s] = pltpu.make_async_copy(
                o_vmem.at[s], out_ref.at[pl.ds(c * CH, CH), :], sem_o.at[s])
            cp_o[s].start()

        for s in range(2):
            if cp_o[s] is not None:
                cp_o[s].wait()
        for c in range(K - 2, K):
            rdmas_q[c].wait_send()
            rdmas_s[c].wait_send()

    out, _recvq, _recvs = pl.pallas_call(
        body,
        out_shape=[
            jax.ShapeDtypeStruct((M_HALF, D), jnp.float32),
            jax.ShapeDtypeStruct((M_HALF, D), jnp.int8),
            jax.ShapeDtypeStruct((M_HALF, 1), jnp.float32),
        ],
        in_specs=[
            pl.BlockSpec(memory_space=pl.ANY),
            pl.BlockSpec(memory_space=pltpu.VMEM),
        ],
        out_specs=[
            pl.BlockSpec(memory_space=pl.ANY),
            pl.BlockSpec(memory_space=pl.ANY),
            pl.BlockSpec(memory_space=pl.ANY),
        ],
        scratch_shapes=[
            pltpu.VMEM((2, CH, D), jnp.float32),
            pltpu.VMEM((2, CH, D), jnp.int8),
            pltpu.VMEM((2, CH, 1), jnp.float32),
            pltpu.VMEM((2, CH, D), jnp.int8),
            pltpu.VMEM((2, CH, 1), jnp.float32),
            pltpu.VMEM((2, CH, D), jnp.float32),
            pltpu.SemaphoreType.DMA((2,)),
            pltpu.SemaphoreType.DMA((2,)),
            pltpu.SemaphoreType.DMA((2,)),
            pltpu.SemaphoreType.DMA((2,)),
            pltpu.SemaphoreType.DMA((2,)),
            pltpu.SemaphoreType.DMA((K,)),
            pltpu.SemaphoreType.DMA((K,)),
            pltpu.SemaphoreType.DMA((K,)),
            pltpu.SemaphoreType.DMA((K,)),
        ],
        compiler_params=pltpu.CompilerParams(
            collective_id=0, vmem_limit_bytes=100 * 1024 * 1024
        ),
    )(partial2, gamma2)
    return out


# device time: 265482 ns/iter; 1.7061x vs baseline; 1.1080x over previous
import jax
import jax.numpy as jnp
from jax import lax
from jax.experimental import pallas as pl
from jax.experimental.pallas import tpu as pltpu

M_GLOBAL = 8192
M_HALF = 4096
D = 4096
CH = 512
K = M_HALF // CH


def kernel(partial, gamma):
    partial2 = partial.reshape(M_GLOBAL, D)
    gamma2 = gamma.reshape(1, D)

    def body(partial_ref, gamma_ref, out_ref, recvq_ref, recvs_ref,
             in_vmem, sendq_vmem, sends_vmem, bq_vmem, bs_vmem, o_vmem,
             sem_in, sem_a, sem_b, sem_bs, sem_o,
             sendq_sems, recvq_sems, sends_sems, recvs_sems):
        a_vmem = in_vmem
        x = lax.axis_index("x")
        y = lax.axis_index("y")
        z = lax.axis_index("z")
        peer = (x, y, 1 - z)

        barrier_sem = pltpu.get_barrier_semaphore()
        pl.semaphore_signal(
            barrier_sem, inc=1,
            device_id=peer, device_id_type=pl.DeviceIdType.MESH,
        )
        pl.semaphore_wait(barrier_sem, 1)

        other_off = (1 - z) * M_HALF
        mine_off = z * M_HALF

        cp_in = [None, None]
        cp_in[0] = pltpu.make_async_copy(
            partial_ref.at[pl.ds(other_off, CH), :],
            in_vmem.at[0], sem_in.at[0])
        cp_in[0].start()
        rdmas_q = []
        rdmas_s = []
        for c in range(K):
            si = c % 2
            if c + 1 < K:
                cp_in[1 - si] = pltpu.make_async_copy(
                    partial_ref.at[pl.ds(other_off + (c + 1) * CH, CH), :],
                    in_vmem.at[1 - si], sem_in.at[1 - si])
                cp_in[1 - si].start()
            cp_in[si].wait()
            xf = in_vmem[si]
            inv = 127.0 / jnp.maximum(
                jnp.max(jnp.abs(xf), axis=1, keepdims=True), 1e-20)
            sendq_vmem[c] = jnp.round(xf * inv).astype(jnp.int8)
            sends_vmem[c] = 1.0 / inv
            rq = pltpu.make_async_remote_copy(
                src_ref=sendq_vmem.at[c],
                dst_ref=recvq_ref.at[pl.ds(c * CH, CH), :],
                send_sem=sendq_sems.at[c],
                recv_sem=recvq_sems.at[c],
                device_id=peer,
                device_id_type=pl.DeviceIdType.MESH,
            )
            rq.start()
            rdmas_q.append(rq)
            rs = pltpu.make_async_remote_copy(
                src_ref=sends_vmem.at[c],
                dst_ref=recvs_ref.at[pl.ds(c * CH, CH), :],
                send_sem=sends_sems.at[c],
                recv_sem=recvs_sems.at[c],
                device_id=peer,
                device_id_type=pl.DeviceIdType.MESH,
            )
            rs.start()
            rdmas_s.append(rs)

        cp_a = [None, None]
        cp_a[0] = pltpu.make_async_copy(
            partial_ref.at[pl.ds(mine_off, CH), :], a_vmem.at[0], sem_a.at[0])
        cp_a[0].start()
        cp_o = [None, None]
        for c in range(K):
            s = c % 2
            if c + 1 < K:
                cp_a[1 - s] = pltpu.make_async_copy(
                    partial_ref.at[pl.ds(mine_off + (c + 1) * CH, CH), :],
                    a_vmem.at[1 - s], sem_a.at[1 - s])
                cp_a[1 - s].start()
            rdmas_q[c].wait_recv()
            rdmas_s[c].wait_recv()
            cp_b = pltpu.make_async_copy(
                recvq_ref.at[pl.ds(c * CH, CH), :], bq_vmem.at[s], sem_b.at[s])
            cp_bs = pltpu.make_async_copy(
                recvs_ref.at[pl.ds(c * CH, CH), :], bs_vmem.at[s],
                sem_bs.at[s])
            cp_b.start()
            cp_bs.start()
            cp_b.wait()
            cp_bs.wait()
            cp_a[s].wait()
            if cp_o[s] is not None:
                cp_o[s].wait()
            b = bq_vmem[s].astype(jnp.float32) * bs_vmem[s]
            ysum = a_vmem[s] + b
            ms = jnp.mean(ysum * ysum, axis=1, keepdims=True)
            o_vmem[s] = ysum * lax.rsqrt(ms + 1e-6) * gamma_ref[...]
            cp_o[s] = pltpu.make_async_copy(
                o_vmem.at[s], out_ref.at[pl.ds(c * CH, CH), :], sem_o.at[s])
            cp_o[s].start()

        for s in range(2):
            if cp_o[s] is not None:
                cp_o[s].wait()
        for c in range(K):
            rdmas_q[c].wait_send()
            rdmas_s[c].wait_send()

    out, _recvq, _recvs = pl.pallas_call(
        body,
        out_shape=[
            jax.ShapeDtypeStruct((M_HALF, D), jnp.float32),
            jax.ShapeDtypeStruct((M_HALF, D), jnp.int8),
            jax.ShapeDtypeStruct((M_HALF, 1), jnp.float32),
        ],
        in_specs=[
            pl.BlockSpec(memory_space=pl.ANY),
            pl.BlockSpec(memory_space=pltpu.VMEM),
        ],
        out_specs=[
            pl.BlockSpec(memory_space=pl.ANY),
            pl.BlockSpec(memory_space=pl.ANY),
            pl.BlockSpec(memory_space=pl.ANY),
        ],
        scratch_shapes=[
            pltpu.VMEM((2, CH, D), jnp.float32),
            pltpu.VMEM((K, CH, D), jnp.int8),
            pltpu.VMEM((K, CH, 1), jnp.float32),
            pltpu.VMEM((2, CH, D), jnp.int8),
            pltpu.VMEM((2, CH, 1), jnp.float32),
            pltpu.VMEM((2, CH, D), jnp.float32),
            pltpu.SemaphoreType.DMA((2,)),
            pltpu.SemaphoreType.DMA((2,)),
            pltpu.SemaphoreType.DMA((2,)),
            pltpu.SemaphoreType.DMA((2,)),
            pltpu.SemaphoreType.DMA((2,)),
            pltpu.SemaphoreType.DMA((K,)),
            pltpu.SemaphoreType.DMA((K,)),
            pltpu.SemaphoreType.DMA((K,)),
            pltpu.SemaphoreType.DMA((K,)),
        ],
        compiler_params=pltpu.CompilerParams(
            collective_id=0, vmem_limit_bytes=100 * 1024 * 1024
        ),
    )(partial2, gamma2)
    return out


# device time: 258436 ns/iter; 1.7526x vs baseline; 1.0273x over previous
import jax
import jax.numpy as jnp
from jax import lax
from jax.experimental import pallas as pl
from jax.experimental.pallas import tpu as pltpu

M_GLOBAL = 8192
M_HALF = 4096
D = 4096
CH = 256
K = M_HALF // CH


def kernel(partial, gamma):
    partial2 = partial.reshape(M_GLOBAL, D)
    gamma2 = gamma.reshape(1, D)

    def body(partial_ref, gamma_ref, out_ref,
             in_vmem, sendq_vmem, sends_vmem, bq_vmem, bs_vmem, o_vmem,
             sem_in, sem_a, sem_o,
             sendq_sems, recvq_sems, sends_sems, recvs_sems):
        a_vmem = in_vmem
        x = lax.axis_index("x")
        y = lax.axis_index("y")
        z = lax.axis_index("z")
        peer = (x, y, 1 - z)

        barrier_sem = pltpu.get_barrier_semaphore()
        pl.semaphore_signal(
            barrier_sem, inc=1,
            device_id=peer, device_id_type=pl.DeviceIdType.MESH,
        )
        pl.semaphore_wait(barrier_sem, 1)

        other_off = (1 - z) * M_HALF
        mine_off = z * M_HALF

        cp_in = [None, None]
        cp_in[0] = pltpu.make_async_copy(
            partial_ref.at[pl.ds(other_off, CH), :],
            in_vmem.at[0], sem_in.at[0])
        cp_in[0].start()
        rdmas_q = []
        rdmas_s = []
        for c in range(K):
            si = c % 2
            if c + 1 < K:
                cp_in[1 - si] = pltpu.make_async_copy(
                    partial_ref.at[pl.ds(other_off + (c + 1) * CH, CH), :],
                    in_vmem.at[1 - si], sem_in.at[1 - si])
                cp_in[1 - si].start()
            cp_in[si].wait()
            xf = in_vmem[si]
            inv = 127.0 / jnp.maximum(
                jnp.max(jnp.abs(xf), axis=1, keepdims=True), 1e-20)
            sendq_vmem[c] = jnp.round(xf * inv).astype(jnp.int8)
            sends_vmem[c] = 1.0 / inv
            rq = pltpu.make_async_remote_copy(
                src_ref=sendq_vmem.at[c],
                dst_ref=bq_vmem.at[c],
                send_sem=sendq_sems.at[c],
                recv_sem=recvq_sems.at[c],
                device_id=peer,
                device_id_type=pl.DeviceIdType.MESH,
            )
            rq.start()
            rdmas_q.append(rq)
            rs = pltpu.make_async_remote_copy(
                src_ref=sends_vmem.at[c],
                dst_ref=bs_vmem.at[c],
                send_sem=sends_sems.at[c],
                recv_sem=recvs_sems.at[c],
                device_id=peer,
                device_id_type=pl.DeviceIdType.MESH,
            )
            rs.start()
            rdmas_s.append(rs)

        cp_a = [None, None]
        cp_a[0] = pltpu.make_async_copy(
            partial_ref.at[pl.ds(mine_off, CH), :], a_vmem.at[0], sem_a.at[0])
        cp_a[0].start()
        cp_o = [None, None]
        for c in range(K):
            s = c % 2
            if c + 1 < K:
                cp_a[1 - s] = pltpu.make_async_copy(
                    partial_ref.at[pl.ds(mine_off + (c + 1) * CH, CH), :],
                    a_vmem.at[1 - s], sem_a.at[1 - s])
                cp_a[1 - s].start()
            rdmas_q[c].wait_recv()
            rdmas_s[c].wait_recv()
            cp_a[s].wait()
            if cp_o[s] is not None:
                cp_o[s].wait()
            b = bq_vmem[c].astype(jnp.float32) * bs_vmem[c]
            ysum = a_vmem[s] + b
            ms = jnp.mean(ysum * ysum, axis=1, keepdims=True)
            o_vmem[s] = ysum * lax.rsqrt(ms + 1e-6) * gamma_ref[...]
            cp_o[s] = pltpu.make_async_copy(
                o_vmem.at[s], out_ref.at[pl.ds(c * CH, CH), :], sem_o.at[s])
            cp_o[s].start()

        for s in range(2):
            if cp_o[s] is not None:
                cp_o[s].wait()
        for c in range(K):
            rdmas_q[c].wait_send()
            rdmas_s[c].wait_send()

    out = pl.pallas_call(
        body,
        out_shape=jax.ShapeDtypeStruct((M_HALF, D), jnp.float32),
        in_specs=[
            pl.BlockSpec(memory_space=pl.ANY),
            pl.BlockSpec(memory_space=pltpu.VMEM),
        ],
        out_specs=pl.BlockSpec(memory_space=pl.ANY),
        scratch_shapes=[
            pltpu.VMEM((2, CH, D), jnp.float32),
            pltpu.VMEM((K, CH, D), jnp.int8),
            pltpu.VMEM((K, CH, 1), jnp.float32),
            pltpu.VMEM((K, CH, D), jnp.int8),
            pltpu.VMEM((K, CH, 1), jnp.float32),
            pltpu.VMEM((2, CH, D), jnp.float32),
            pltpu.SemaphoreType.DMA((2,)),
            pltpu.SemaphoreType.DMA((2,)),
            pltpu.SemaphoreType.DMA((2,)),
            pltpu.SemaphoreType.DMA((K,)),
            pltpu.SemaphoreType.DMA((K,)),
            pltpu.SemaphoreType.DMA((K,)),
            pltpu.SemaphoreType.DMA((K,)),
        ],
        compiler_params=pltpu.CompilerParams(
            collective_id=0, vmem_limit_bytes=100 * 1024 * 1024
        ),
    )(partial2, gamma2)
    return out
